# baseline (speedup 1.0000x reference)
import functools

import jax
import jax.numpy as jnp
from jax import lax
from jax.experimental import pallas as pl
from jax.experimental.pallas import tpu as pltpu
from jax.experimental.pallas import tpu_sc as plsc

BATCH = 16384
VOCAB = 100000
HIER_VOCAB = 10000
EMBED_DIM = 32
OUT_DIM = 32
PAD_DIM = 128

_NC = 2
_NS = 16
_NW = _NC * _NS
_B_PER_W = BATCH // _NW
_CHUNK = 128
_NCHUNK = _B_PER_W // _CHUNK


def _sc_gather_body(idx_hbm, par_hbm, code_hbm, hier_hbm, ec_out, ep_out,
                    idx_v, par_v, c0, c1, h0, h1,
                    sc0, sc1, sh0, sh1):
    wid = lax.axis_index("s") * _NC + lax.axis_index("c")
    base = wid * _B_PER_W
    for j in range(_NCHUNK):
        pltpu.sync_copy(idx_hbm.at[pl.ds(base + j * _CHUNK, _CHUNK)], idx_v.at[j])
        pltpu.sync_copy(par_hbm.at[pl.ds(base + j * _CHUNK, _CHUNK)], par_v.at[j])
    cbufs = (c0, c1)
    hbufs = (h0, h1)
    csems = (sc0, sc1)
    hsems = (sh0, sh1)
    cps_c = [None] * _NCHUNK
    cps_h = [None] * _NCHUNK
    for j in range(2):
        cps_c[j] = pltpu.async_copy(code_hbm.at[idx_v.at[j]], cbufs[j], csems[j])
        cps_h[j] = pltpu.async_copy(hier_hbm.at[par_v.at[j]], hbufs[j], hsems[j])
    for j in range(_NCHUNK):
        dst = pl.ds(base + j * _CHUNK, _CHUNK)
        cps_c[j].wait()
        pltpu.sync_copy(cbufs[j % 2], ec_out.at[dst])
        if j + 2 < _NCHUNK:
            cps_c[j + 2] = pltpu.async_copy(
                code_hbm.at[idx_v.at[j + 2]], cbufs[j % 2], csems[j % 2])
        cps_h[j].wait()
        pltpu.sync_copy(hbufs[j % 2], ep_out.at[dst])
        if j + 2 < _NCHUNK:
            cps_h[j + 2] = pltpu.async_copy(
                hier_hbm.at[par_v.at[j + 2]], hbufs[j % 2], hsems[j % 2])


_sc_gather = functools.partial(
    pl.kernel,
    out_type=[
        jax.ShapeDtypeStruct((BATCH, PAD_DIM), jnp.float32),
        jax.ShapeDtypeStruct((BATCH, PAD_DIM), jnp.float32),
    ],
    mesh=plsc.VectorSubcoreMesh(core_axis_name="c", subcore_axis_name="s"),
    compiler_params=pltpu.CompilerParams(use_tc_tiling_on_sc=False),
    scratch_types=[
        pltpu.VMEM((_NCHUNK, _CHUNK), jnp.int32),
        pltpu.VMEM((_NCHUNK, _CHUNK), jnp.int32),
        pltpu.VMEM((_CHUNK, PAD_DIM), jnp.float32),
        pltpu.VMEM((_CHUNK, PAD_DIM), jnp.float32),
        pltpu.VMEM((_CHUNK, PAD_DIM), jnp.float32),
        pltpu.VMEM((_CHUNK, PAD_DIM), jnp.float32),
        pltpu.SemaphoreType.DMA,
        pltpu.SemaphoreType.DMA,
        pltpu.SemaphoreType.DMA,
        pltpu.SemaphoreType.DMA,
    ],
)(_sc_gather_body)


_TP_BLK = 512


def _tp_body(tt_ref, o_ref):
    t = jax.lax.dot_general(
        tt_ref[...], jnp.eye(EMBED_DIM, dtype=jnp.float32),
        (((0,), (0,)), ((), ())), preferred_element_type=jnp.float32)
    o_ref[...] = jnp.concatenate(
        [t, jnp.zeros((_TP_BLK, PAD_DIM - EMBED_DIM), jnp.float32)], axis=1)


def _tc_transpose_pad(table_t, vocab):
    grid = ((vocab + _TP_BLK - 1) // _TP_BLK,)
    return pl.pallas_call(
        _tp_body,
        grid=grid,
        in_specs=[pl.BlockSpec((EMBED_DIM, _TP_BLK), lambda i: (0, i))],
        out_specs=pl.BlockSpec((_TP_BLK, PAD_DIM), lambda i: (i, 0)),
        out_shape=jax.ShapeDtypeStruct((vocab, PAD_DIM), jnp.float32),
    )(table_t)


_MM_BLK = 2048


def _mm_body(ec_ref, ep_ref, w1_ref, w2_ref, b_ref, o_ref):
    acc = jax.lax.dot_general(
        w1_ref[...], ec_ref[...], (((0,), (1,)), ((), ())),
        preferred_element_type=jnp.float32)
    acc += jax.lax.dot_general(
        w2_ref[...], ep_ref[...], (((0,), (1,)), ((), ())),
        preferred_element_type=jnp.float32)
    o_ref[...] = acc + b_ref[...]


def _tc_project(ec, ep, w1p, w2p, bcol):
    grid = (BATCH // _MM_BLK,)
    return pl.pallas_call(
        _mm_body,
        grid=grid,
        in_specs=[
            pl.BlockSpec((_MM_BLK, PAD_DIM), lambda i: (i, 0)),
            pl.BlockSpec((_MM_BLK, PAD_DIM), lambda i: (i, 0)),
            pl.BlockSpec((PAD_DIM, OUT_DIM), lambda i: (0, 0)),
            pl.BlockSpec((PAD_DIM, OUT_DIM), lambda i: (0, 0)),
            pl.BlockSpec((OUT_DIM, 1), lambda i: (0, 0)),
        ],
        out_specs=pl.BlockSpec((OUT_DIM, _MM_BLK), lambda i: (0, i)),
        out_shape=jax.ShapeDtypeStruct((OUT_DIM, BATCH), jnp.float32),
    )(ec, ep, w1p, w2p, bcol)


@jax.jit
def kernel(indices, parents, code_table, hier_table, W, b):
    pad = PAD_DIM - EMBED_DIM
    code_pad = _tc_transpose_pad(code_table.T, VOCAB)
    hier_pad = _tc_transpose_pad(hier_table.T, HIER_VOCAB)
    ec, ep = _sc_gather(indices, parents, code_pad, hier_pad)
    w1p = jnp.pad(W[:EMBED_DIM], ((0, pad), (0, 0)))
    w2p = jnp.pad(W[EMBED_DIM:], ((0, pad), (0, 0)))
    out_t = _tc_project(ec, ep, w1p, w2p, b.reshape(OUT_DIM, 1))
    return out_t.T


# speedup vs baseline: 2.2535x; 2.2535x over previous
import functools

import jax
import jax.numpy as jnp
from jax import lax
from jax.experimental import pallas as pl
from jax.experimental.pallas import tpu as pltpu
from jax.experimental.pallas import tpu_sc as plsc

BATCH = 16384
VOCAB = 100000
HIER_VOCAB = 10000
EMBED_DIM = 32
OUT_DIM = 32
PAD_DIM = 128

_NC = 2
_NS = 16
_NW = _NC * _NS
_B_PER_W = BATCH // _NW
_CHUNK = 128
_NCHUNK = _B_PER_W // _CHUNK


def _sc_gather_body(idx_hbm, par_hbm, code_hbm, hier_hbm, ec_out, ep_out,
                    idx_v, par_v, c0, c1, h0, h1,
                    sc0, sc1, sh0, sh1):
    wid = lax.axis_index("s") * _NC + lax.axis_index("c")
    base = wid * _B_PER_W
    for j in range(_NCHUNK):
        pltpu.sync_copy(idx_hbm.at[pl.ds(base + j * _CHUNK, _CHUNK)], idx_v.at[j])
        pltpu.sync_copy(par_hbm.at[pl.ds(base + j * _CHUNK, _CHUNK)], par_v.at[j])
    cbufs = (c0, c1)
    hbufs = (h0, h1)
    csems = (sc0, sc1)
    hsems = (sh0, sh1)
    cps_c = [None] * _NCHUNK
    cps_h = [None] * _NCHUNK
    for j in range(2):
        cps_c[j] = pltpu.async_copy(code_hbm.at[idx_v.at[j]], cbufs[j], csems[j])
        cps_h[j] = pltpu.async_copy(hier_hbm.at[par_v.at[j]], hbufs[j], hsems[j])
    for j in range(_NCHUNK):
        dst = pl.ds(base + j * _CHUNK, _CHUNK)
        cps_c[j].wait()
        pltpu.sync_copy(cbufs[j % 2], ec_out.at[dst])
        if j + 2 < _NCHUNK:
            cps_c[j + 2] = pltpu.async_copy(
                code_hbm.at[idx_v.at[j + 2]], cbufs[j % 2], csems[j % 2])
        cps_h[j].wait()
        pltpu.sync_copy(hbufs[j % 2], ep_out.at[dst])
        if j + 2 < _NCHUNK:
            cps_h[j + 2] = pltpu.async_copy(
                hier_hbm.at[par_v.at[j + 2]], hbufs[j % 2], hsems[j % 2])


_sc_gather = functools.partial(
    pl.kernel,
    out_type=[
        jax.ShapeDtypeStruct((BATCH, PAD_DIM), jnp.float32),
        jax.ShapeDtypeStruct((BATCH, PAD_DIM), jnp.float32),
    ],
    mesh=plsc.VectorSubcoreMesh(core_axis_name="c", subcore_axis_name="s"),
    compiler_params=pltpu.CompilerParams(use_tc_tiling_on_sc=False),
    scratch_types=[
        pltpu.VMEM((_NCHUNK, _CHUNK), jnp.int32),
        pltpu.VMEM((_NCHUNK, _CHUNK), jnp.int32),
        pltpu.VMEM((_CHUNK, PAD_DIM), jnp.float32),
        pltpu.VMEM((_CHUNK, PAD_DIM), jnp.float32),
        pltpu.VMEM((_CHUNK, PAD_DIM), jnp.float32),
        pltpu.VMEM((_CHUNK, PAD_DIM), jnp.float32),
        pltpu.SemaphoreType.DMA,
        pltpu.SemaphoreType.DMA,
        pltpu.SemaphoreType.DMA,
        pltpu.SemaphoreType.DMA,
    ],
)(_sc_gather_body)


_TP_BLK = 4096


def _tp_body(tt_ref, o_ref):
    t = jax.lax.dot_general(
        tt_ref[...], jnp.eye(EMBED_DIM, dtype=jnp.float32),
        (((0,), (0,)), ((), ())), preferred_element_type=jnp.float32)
    o_ref[...] = jnp.concatenate(
        [t, jnp.zeros((_TP_BLK, PAD_DIM - EMBED_DIM), jnp.float32)], axis=1)


def _tc_transpose_pad(table_t, vocab):
    grid = ((vocab + _TP_BLK - 1) // _TP_BLK,)
    return pl.pallas_call(
        _tp_body,
        grid=grid,
        in_specs=[pl.BlockSpec((EMBED_DIM, _TP_BLK), lambda i: (0, i))],
        out_specs=pl.BlockSpec((_TP_BLK, PAD_DIM), lambda i: (i, 0)),
        out_shape=jax.ShapeDtypeStruct((vocab, PAD_DIM), jnp.float32),
    )(table_t)


_MM_BLK = 2048


def _mm_body(ec_ref, ep_ref, w1_ref, w2_ref, b_ref, o_ref):
    acc = jax.lax.dot_general(
        w1_ref[...], ec_ref[...], (((0,), (1,)), ((), ())),
        preferred_element_type=jnp.float32)
    acc += jax.lax.dot_general(
        w2_ref[...], ep_ref[...], (((0,), (1,)), ((), ())),
        preferred_element_type=jnp.float32)
    o_ref[...] = acc + b_ref[...]


def _tc_project(ec, ep, w1p, w2p, bcol):
    grid = (BATCH // _MM_BLK,)
    return pl.pallas_call(
        _mm_body,
        grid=grid,
        in_specs=[
            pl.BlockSpec((_MM_BLK, PAD_DIM), lambda i: (i, 0)),
            pl.BlockSpec((_MM_BLK, PAD_DIM), lambda i: (i, 0)),
            pl.BlockSpec((PAD_DIM, OUT_DIM), lambda i: (0, 0)),
            pl.BlockSpec((PAD_DIM, OUT_DIM), lambda i: (0, 0)),
            pl.BlockSpec((OUT_DIM, 1), lambda i: (0, 0)),
        ],
        out_specs=pl.BlockSpec((OUT_DIM, _MM_BLK), lambda i: (0, i)),
        out_shape=jax.ShapeDtypeStruct((OUT_DIM, BATCH), jnp.float32),
    )(ec, ep, w1p, w2p, bcol)


@jax.jit
def kernel(indices, parents, code_table, hier_table, W, b):
    pad = PAD_DIM - EMBED_DIM
    code_pad = _tc_transpose_pad(code_table.T, VOCAB)
    hier_pad = _tc_transpose_pad(hier_table.T, HIER_VOCAB)
    ec, ep = _sc_gather(indices, parents, code_pad, hier_pad)
    w1p = jnp.pad(W[:EMBED_DIM], ((0, pad), (0, 0)))
    w2p = jnp.pad(W[EMBED_DIM:], ((0, pad), (0, 0)))
    out_t = _tc_project(ec, ep, w1p, w2p, b.reshape(OUT_DIM, 1))
    return out_t.T
